# Initial kernel scaffold; baseline (speedup 1.0000x reference)
import jax
import jax.numpy as jnp
from jax.experimental import pallas as pl


def kernel(x, edge_index, edge_attr, batch, node_w, node_b, edge_w, edge_b, mlp_w1, mlp_b1, mlp_w2, mlp_b2, eps, bn_g, bn_b, head_w1, head_b1, head_w2, head_b2):
    raise NotImplementedError("write your pallas kernel here")


# speedup vs baseline: 1.2849x; 1.2849x over previous
import jax
import jax.numpy as jnp
from jax import lax
from jax.experimental import pallas as pl
from jax.experimental.pallas import tpu as pltpu
from jax.experimental.pallas import tpu_sc as plsc

N = 50000
E = 800000
G = 64
ND = 28
ED = 7
H = 128
L = 3
Q = 32

NC = 2
NS = 16
NW = NC * NS
NP = 50176
RPT = NP // NS

E2 = 802816
EW = E2 // NW
EB1 = 3136
NBLK1 = EW // EB1
C1 = 112
NCH1 = EB1 // C1

EC = E // NS
EB2 = 2000
NBLK2 = EC // EB2
C2 = 80
NCH2 = EB2 // C2

_SC_MESH = dict(core_axis_name="c", subcore_axis_name="s")


def _msg_body(h_hbm, e_hbm, src_hbm, m_hbm,
              srcv, sidx, bufE, bufH, bm0, bm1, bm2, bm3, semE, semH, semW):
    c = lax.axis_index("c")
    s = lax.axis_index("s")
    w = s * NC + c
    bufM = (bm0, bm1, bm2, bm3)

    def _block(b, _):
        base = w * EW + b * EB1
        pltpu.sync_copy(src_hbm.at[pl.ds(base, EB1)], srcv)

        def _chunk(ci, _):
            o0 = ci * C1
            for j in range(C1 // 16):
                sidx[pl.ds(j * 16, 16)] = srcv[pl.ds(o0 + j * 16, 16)]
            cpe = pltpu.async_copy(e_hbm.at[pl.ds(base + o0, C1)], bufE, semE)
            cph = pltpu.async_copy(h_hbm.at[sidx], bufH, semH)
            cpe.wait()
            cph.wait()

            def _relu(rr, _):
                for dr in range(4):
                    r = rr * 4 + dr
                    for j in range(8):
                        v = bufE[r, pl.ds(j * 16, 16)] + \
                            bufH[r, pl.ds(j * 16, 16)]
                        bufM[j // 2][r, pl.ds((j % 2) * 16, 16)] = \
                            jnp.maximum(v, 0.0)
                return 0
            lax.fori_loop(0, C1 // 4, _relu, 0)
            cps = [pltpu.async_copy(
                bufM[qq], m_hbm.at[pl.ds(qq * E2 + base + o0, C1)], semW)
                for qq in range(4)]
            for cp in cps:
                cp.wait()
            return 0
        lax.fori_loop(0, NCH1, _chunk, 0)
        return 0
    lax.fori_loop(0, NBLK1, _block, 0)


def _messages(h, e2, src2):
    return pl.kernel(
        _msg_body,
        out_type=jax.ShapeDtypeStruct((4 * E2, Q), jnp.float32),
        mesh=plsc.VectorSubcoreMesh(**_SC_MESH),
        scratch_types=[
            pltpu.VMEM((EB1,), jnp.int32),
            pltpu.VMEM((C1,), jnp.int32),
            pltpu.VMEM((C1, H), jnp.float32),
            pltpu.VMEM((C1, H), jnp.float32),
            pltpu.VMEM((C1, Q), jnp.float32),
            pltpu.VMEM((C1, Q), jnp.float32),
            pltpu.VMEM((C1, Q), jnp.float32),
            pltpu.VMEM((C1, Q), jnp.float32),
            pltpu.SemaphoreType.DMA,
            pltpu.SemaphoreType.DMA,
            pltpu.SemaphoreType.DMA,
        ],
    )(h, e2, src2)


_ZC = 64
_NZC = RPT // _ZC


def _scat_body(m_hbm, dst_hbm, aggr_hbm,
               acc, dstv, didx, zidx, bufA, bufZ, semM):
    c = lax.axis_index("c")
    s = lax.axis_index("s")
    iota16 = lax.iota(jnp.int32, 16)
    zero16 = jnp.zeros((16,), jnp.float32)

    def _zb(i, _):
        for j in range(Q // 16):
            bufZ[i, pl.ds(j * 16, 16)] = zero16
        return 0
    lax.fori_loop(0, _ZC, _zb, 0)

    for p in range(2):
        q = 2 * c + p
        def _zero(t, _):
            for j in range(_ZC // 16):
                zidx[pl.ds(j * 16, 16)] = iota16 + (s * RPT + t * _ZC + j * 16)
            pltpu.sync_copy(bufZ, acc.at[zidx])
            return 0
        lax.fori_loop(0, _NZC, _zero, 0)
        plsc.subcore_barrier()

        def _block(b, _):
            base = s * EC + b * EB2
            pltpu.sync_copy(dst_hbm.at[pl.ds(base, EB2)], dstv)

            def _chunk(ci, _):
                o0 = ci * C2
                for j in range(C2 // 16):
                    didx[pl.ds(j * 16, 16)] = dstv[pl.ds(o0 + j * 16, 16)]
                pltpu.async_copy(
                    m_hbm.at[pl.ds(q * E2 + base + o0, C2)], bufA, semM).wait()
                pltpu.sync_copy(bufA, acc.at[didx], add=True)
                return 0
            lax.fori_loop(0, NCH2, _chunk, 0)
            return 0
        lax.fori_loop(0, NBLK2, _block, 0)

        plsc.subcore_barrier()

        def _dump(t, _):
            for j in range(_ZC // 16):
                zidx[pl.ds(j * 16, 16)] = iota16 + (s * RPT + t * _ZC + j * 16)
            pltpu.sync_copy(acc.at[zidx], bufA.at[pl.ds(0, _ZC)])
            pltpu.sync_copy(
                bufA.at[pl.ds(0, _ZC)],
                aggr_hbm.at[pl.ds(q * NP + s * RPT + t * _ZC, _ZC)])
            return 0
        lax.fori_loop(0, _NZC, _dump, 0)
        plsc.subcore_barrier()


def _scatter_agg(mq, dst):
    return pl.kernel(
        _scat_body,
        out_type=jax.ShapeDtypeStruct((4 * NP, Q), jnp.float32),
        mesh=plsc.VectorSubcoreMesh(**_SC_MESH),
        scratch_types=[
            pltpu.VMEM_SHARED((NP, Q), jnp.float32),
            pltpu.VMEM((EB2,), jnp.int32),
            pltpu.VMEM((C2,), jnp.int32),
            pltpu.VMEM((_ZC,), jnp.int32),
            pltpu.VMEM((C2, Q), jnp.float32),
            pltpu.VMEM((_ZC, Q), jnp.float32),
            pltpu.SemaphoreType.DMA,
        ],
    )(mq, dst)


_PRW = 1568
_PRB = 112


def _pool_body(h_hbm, batch_hbm, psum_hbm, pmax_hbm, pcnt_hbm,
               hrows, gvec, asum, amax, acnt):
    c = lax.axis_index("c")
    s = lax.axis_index("s")
    w = s * NC + c
    start = w * _PRW
    nrows = jnp.maximum(0, jnp.minimum(_PRW, N - start))
    iota16 = lax.iota(jnp.int32, 16)
    zero16 = jnp.zeros((16,), jnp.float32)
    ninf16 = jnp.full((16,), -jnp.inf, jnp.float32)

    def _zi(i, _):
        for j in range(8):
            asum[i, pl.ds(j * 16, 16)] = zero16
            amax[i, pl.ds(j * 16, 16)] = ninf16
        return 0
    lax.fori_loop(0, G + 1, _zi, 0)
    for i in range(8):
        acnt[pl.ds(i * 16, 16)] = zero16

    nblk = (nrows + (_PRB - 1)) // _PRB

    def _blk(bi, _):
        rbase = start + bi * _PRB
        nr = nrows - bi * _PRB
        pltpu.sync_copy(h_hbm.at[pl.ds(rbase, _PRB)], hrows)
        pltpu.sync_copy(batch_hbm.at[pl.ds(rbase, _PRB)], gvec)

        def _rowgrp(gi, _):
            gv = gvec[pl.ds(gi * 16, 16)]
            for lj in range(16):
                r = gi * 16 + lj
                g = jnp.where(r < nr, gv[lj], G)
                for j in range(8):
                    v = hrows[r, pl.ds(j * 16, 16)]
                    plsc.addupdate(asum.at[g, pl.ds(j * 16, 16)], v)
                    cur = amax[g, pl.ds(j * 16, 16)]
                    amax[g, pl.ds(j * 16, 16)] = jnp.maximum(cur, v)
                gq = (g // 16) * 16
                one = jnp.where(iota16 == (g - gq), 1.0, 0.0)
                plsc.addupdate(acnt.at[pl.ds(gq, 16)], one)
            return 0
        lax.fori_loop(0, _PRB // 16, _rowgrp, 0)
        return 0
    lax.fori_loop(0, nblk, _blk, 0)

    pltpu.sync_copy(asum.at[pl.ds(0, G)], psum_hbm.at[w])
    pltpu.sync_copy(amax.at[pl.ds(0, G)], pmax_hbm.at[w])
    pltpu.sync_copy(acnt, pcnt_hbm.at[w])


def _pool_partials(h, batch_pad):
    return pl.kernel(
        _pool_body,
        out_type=(
            jax.ShapeDtypeStruct((NW, G, H), jnp.float32),
            jax.ShapeDtypeStruct((NW, G, H), jnp.float32),
            jax.ShapeDtypeStruct((NW, H), jnp.float32),
        ),
        mesh=plsc.VectorSubcoreMesh(**_SC_MESH),
        scratch_types=[
            pltpu.VMEM((_PRB, H), jnp.float32),
            pltpu.VMEM((_PRB,), jnp.int32),
            pltpu.VMEM((G + 1, H), jnp.float32),
            pltpu.VMEM((G + 1, H), jnp.float32),
            pltpu.VMEM((H,), jnp.float32),
        ],
    )(h, batch_pad)


_MB = 512
_NBLOCKS = NP // _MB


def _proj_kernel(x_ref, w_ref, b_ref, o_ref):
    o_ref[...] = jnp.dot(x_ref[...], w_ref[...],
                         preferred_element_type=jnp.float32) + b_ref[...]


def _project(x, w, b, rows_per_block):
    n = x.shape[0]
    d = x.shape[1]
    grid = n // rows_per_block
    return pl.pallas_call(
        _proj_kernel,
        grid=(grid,),
        in_specs=[
            pl.BlockSpec((rows_per_block, d), lambda i: (i, 0)),
            pl.BlockSpec((d, H), lambda i: (0, 0)),
            pl.BlockSpec((1, H), lambda i: (0, 0)),
        ],
        out_specs=pl.BlockSpec((rows_per_block, H), lambda i: (i, 0)),
        out_shape=jax.ShapeDtypeStruct((n, H), jnp.float32),
    )(x, w, b.reshape(1, H))


def _mlp_kernel(h_ref, a_ref, w1_ref, b1_ref, w2_ref, b2_ref, eps_ref,
                z_ref, sums_ref):
    pid = pl.program_id(0)
    acat = jnp.concatenate([a_ref[q] for q in range(4)], axis=-1)
    u = h_ref[...] * eps_ref[...] + acat
    t = jnp.maximum(jnp.dot(u, w1_ref[...],
                            preferred_element_type=jnp.float32) + b1_ref[...], 0.0)
    z = jnp.dot(t, w2_ref[...], preferred_element_type=jnp.float32) + b2_ref[...]
    z_ref[...] = z
    row = pid * _MB + lax.broadcasted_iota(jnp.int32, (_MB, 1), 0)
    zm = jnp.where(row < N, z, 0.0)
    part = jnp.concatenate(
        [jnp.sum(zm, axis=0, keepdims=True),
         jnp.sum(zm * zm, axis=0, keepdims=True)], axis=0)

    @pl.when(pid == 0)
    def _():
        sums_ref[...] = part

    @pl.when(pid > 0)
    def _():
        sums_ref[...] = sums_ref[...] + part


def _mlp_stats(h, aggrq, w1, b1, w2, b2, epsl):
    epsv = jnp.full((1, H), 1.0 + epsl, jnp.float32)
    a4 = aggrq.reshape(4, NP, Q)
    return pl.pallas_call(
        _mlp_kernel,
        grid=(_NBLOCKS,),
        in_specs=[
            pl.BlockSpec((_MB, H), lambda i: (i, 0)),
            pl.BlockSpec((4, _MB, Q), lambda i: (0, i, 0)),
            pl.BlockSpec((H, H), lambda i: (0, 0)),
            pl.BlockSpec((1, H), lambda i: (0, 0)),
            pl.BlockSpec((H, H), lambda i: (0, 0)),
            pl.BlockSpec((1, H), lambda i: (0, 0)),
            pl.BlockSpec((1, H), lambda i: (0, 0)),
        ],
        out_specs=[
            pl.BlockSpec((_MB, H), lambda i: (i, 0)),
            pl.BlockSpec((2, H), lambda i: (0, 0)),
        ],
        out_shape=[
            jax.ShapeDtypeStruct((NP, H), jnp.float32),
            jax.ShapeDtypeStruct((2, H), jnp.float32),
        ],
    )(h, a4, w1, b1.reshape(1, H), w2, b2.reshape(1, H), epsv)


def _bn_kernel(z_ref, h_ref, sums_ref, g_ref, b_ref, o_ref):
    mu = sums_ref[0:1, :] * (1.0 / N)
    var = sums_ref[1:2, :] * (1.0 / N) - mu * mu
    inv = lax.rsqrt(var + 1e-5) * g_ref[...]
    zn = (z_ref[...] - mu) * inv + b_ref[...]
    o_ref[...] = h_ref[...] + jnp.maximum(zn, 0.0)


def _bn_residual(z, h, sums, g, b):
    return pl.pallas_call(
        _bn_kernel,
        grid=(_NBLOCKS,),
        in_specs=[
            pl.BlockSpec((_MB, H), lambda i: (i, 0)),
            pl.BlockSpec((_MB, H), lambda i: (i, 0)),
            pl.BlockSpec((2, H), lambda i: (0, 0)),
            pl.BlockSpec((1, H), lambda i: (0, 0)),
            pl.BlockSpec((1, H), lambda i: (0, 0)),
        ],
        out_specs=pl.BlockSpec((_MB, H), lambda i: (i, 0)),
        out_shape=jax.ShapeDtypeStruct((NP, H), jnp.float32),
    )(z, h, sums, g.reshape(1, H), b.reshape(1, H))


def _head_kernel(ps_ref, pm_ref, pc_ref, w1_ref, b1_ref, w2_ref, b2_ref,
                 o_ref):
    ssum = jnp.sum(ps_ref[...], axis=0)
    smax = jnp.max(pm_ref[...], axis=0)
    cnt = jnp.sum(pc_ref[...], axis=0)[:G, None]
    mean = ssum / jnp.maximum(cnt, 1.0)
    p = jnp.concatenate([mean, smax], axis=-1)
    t = jnp.maximum(jnp.dot(p, w1_ref[...],
                            preferred_element_type=jnp.float32) + b1_ref[...], 0.0)
    res = jnp.sum(t * w2_ref[...], axis=1) + b2_ref[0, 0]
    o_ref[0, :] = res


def _head(psum, pmax, pcnt, hw1, hb1, hw2, hb2):
    return pl.pallas_call(
        _head_kernel,
        in_specs=[
            pl.BlockSpec((NW, G, H), lambda: (0, 0, 0)),
            pl.BlockSpec((NW, G, H), lambda: (0, 0, 0)),
            pl.BlockSpec((NW, H), lambda: (0, 0)),
            pl.BlockSpec((2 * H, H), lambda: (0, 0)),
            pl.BlockSpec((1, H), lambda: (0, 0)),
            pl.BlockSpec((1, H), lambda: (0, 0)),
            pl.BlockSpec((1, 1), lambda: (0, 0)),
        ],
        out_specs=pl.BlockSpec((1, G), lambda: (0, 0)),
        out_shape=jax.ShapeDtypeStruct((1, G), jnp.float32),
    )(psum, pmax, pcnt, hw1, hb1.reshape(1, H), hw2.reshape(1, H),
      hb2.reshape(1, 1))


def kernel(x, edge_index, edge_attr, batch, node_w, node_b, edge_w, edge_b,
           mlp_w1, mlp_b1, mlp_w2, mlp_b2, eps, bn_g, bn_b,
           head_w1, head_b1, head_w2, head_b2):
    x_pad = jnp.concatenate([x, jnp.zeros((NP - N, ND), jnp.float32)], axis=0)
    batch_pad = jnp.concatenate([batch, jnp.zeros((NP - N,), jnp.int32)])
    ea_pad = jnp.concatenate(
        [edge_attr, jnp.zeros((E2 - E, ED), jnp.float32)], axis=0)
    src2 = jnp.concatenate(
        [edge_index[0], jnp.zeros((E2 - E,), jnp.int32)])
    dst = edge_index[1]

    h = _project(x_pad, node_w, node_b, 448)
    e2 = _project(ea_pad, edge_w, edge_b, 6272)

    for l in range(L):
        mq = _messages(h, e2, src2)
        aggrq = _scatter_agg(mq, dst)
        z, sums = _mlp_stats(h, aggrq, mlp_w1[l], mlp_b1[l], mlp_w2[l],
                             mlp_b2[l], eps[l])
        h = _bn_residual(z, h, sums, bn_g[l], bn_b[l])

    psum, pmax, pcnt = _pool_partials(h, batch_pad)
    out = _head(psum, pmax, pcnt, head_w1, head_b1, head_w2, head_b2)
    return out.reshape(G)
